# Initial kernel scaffold; baseline (speedup 1.0000x reference)
import jax
import jax.numpy as jnp
from jax.experimental import pallas as pl


def kernel(x, edge_index, W1, al1, ar1, b1, W2, al2, ar2, b2):
    raise NotImplementedError("write your pallas kernel here")


# speedup vs baseline: 2.0316x; 2.0316x over previous
import functools

import jax
import jax.numpy as jnp
from jax.experimental import pallas as pl

_N = 10000
_E = 160000
_NODE_TILE = 400
_EDGE_TILE = 640


def _proj_body(x_ref, b_ref, w_ref, a_ref, feat_ref, eler_ref, *, apply_relu):
    xb = x_ref[...] + b_ref[...]
    if apply_relu:
        xb = jnp.maximum(xb, 0.0)
    feat = jnp.dot(xb, w_ref[...], preferred_element_type=jnp.float32)
    feat_ref[...] = feat
    eler_ref[...] = jnp.dot(feat, a_ref[...], preferred_element_type=jnp.float32)


def _proj(x, b, w, a_mat, apply_relu):
    n, k = x.shape
    m = w.shape[1]
    na = a_mat.shape[1]
    grid = n // _NODE_TILE
    return pl.pallas_call(
        functools.partial(_proj_body, apply_relu=apply_relu),
        grid=(grid,),
        in_specs=[
            pl.BlockSpec((_NODE_TILE, k), lambda i: (i, 0)),
            pl.BlockSpec((1, k), lambda i: (0, 0)),
            pl.BlockSpec((k, m), lambda i: (0, 0)),
            pl.BlockSpec((m, na), lambda i: (0, 0)),
        ],
        out_specs=[
            pl.BlockSpec((_NODE_TILE, m), lambda i: (i, 0)),
            pl.BlockSpec((_NODE_TILE, na), lambda i: (i, 0)),
        ],
        out_shape=[
            jax.ShapeDtypeStruct((n, m), jnp.float32),
            jax.ShapeDtypeStruct((n, na), jnp.float32),
        ],
    )(x, b.reshape(1, k), w, a_mat)


def _edge_logit_body(els_ref, erd_ref, e_ref):
    s = els_ref[...] + erd_ref[...]
    e_ref[...] = jnp.where(s >= 0.0, s, 0.2 * s)


def _edge_exp_body(e_ref, m_ref, ex_ref):
    m = m_ref[...]
    m = jnp.where(jnp.isfinite(m), m, 0.0)
    ex_ref[...] = jnp.exp(e_ref[...] - m)


def _edge_ew(body, a, b):
    e, h = a.shape
    grid = e // _EDGE_TILE
    spec = pl.BlockSpec((_EDGE_TILE, h), lambda i: (i, 0))
    return pl.pallas_call(
        body,
        grid=(grid,),
        in_specs=[spec, spec],
        out_specs=spec,
        out_shape=jax.ShapeDtypeStruct((e, h), jnp.float32),
    )(a, b)


def _msg_body(ex_ref, den_ref, fs_ref, exp_ref, msg_ref):
    alpha = ex_ref[...] / jnp.maximum(den_ref[...], 1e-9)
    a_full = jnp.dot(alpha, exp_ref[...], preferred_element_type=jnp.float32)
    msg_ref[...] = a_full * fs_ref[...]


def _messages(ex, den_dst, feat_src, expand):
    e, h = ex.shape
    m = feat_src.shape[1]
    grid = e // _EDGE_TILE
    small = pl.BlockSpec((_EDGE_TILE, h), lambda i: (i, 0))
    big = pl.BlockSpec((_EDGE_TILE, m), lambda i: (i, 0))
    return pl.pallas_call(
        _msg_body,
        grid=(grid,),
        in_specs=[small, small, big, pl.BlockSpec((h, m), lambda i: (0, 0))],
        out_specs=big,
        out_shape=jax.ShapeDtypeStruct((e, m), jnp.float32),
    )(ex, den_dst, feat_src, expand)


def _bias_relu_body(x_ref, b_ref, o_ref):
    o_ref[...] = jnp.maximum(x_ref[...] + b_ref[...], 0.0)


def _bias_relu(x, b):
    n, m = x.shape
    grid = n // _NODE_TILE
    return pl.pallas_call(
        _bias_relu_body,
        grid=(grid,),
        in_specs=[
            pl.BlockSpec((_NODE_TILE, m), lambda i: (i, 0)),
            pl.BlockSpec((1, m), lambda i: (0, 0)),
        ],
        out_specs=pl.BlockSpec((_NODE_TILE, m), lambda i: (i, 0)),
        out_shape=jax.ShapeDtypeStruct((n, m), jnp.float32),
    )(x, b.reshape(1, m))


def _gat_layer(x, src, dst, w, al, ar, b, h, d, in_bias, in_relu):
    n = x.shape[0]
    hd = h * d
    a_mat = jnp.zeros((hd, 2 * h), dtype=jnp.float32)
    heads = jnp.arange(hd, dtype=jnp.int32) // d
    a_mat = a_mat.at[jnp.arange(hd), heads].set(al.reshape(-1))
    a_mat = a_mat.at[jnp.arange(hd), h + heads].set(ar.reshape(-1))
    feat, eler = _proj(x, in_bias, w, a_mat, in_relu)
    el = eler[:, :h]
    er = eler[:, h:]

    e = _edge_ew(_edge_logit_body, jnp.take(el, src, axis=0),
                 jnp.take(er, dst, axis=0))
    m = jax.ops.segment_max(e, dst, num_segments=n)
    ex = _edge_ew(_edge_exp_body, e, jnp.take(m, dst, axis=0))
    den = jax.ops.segment_sum(ex, dst, num_segments=n)

    expand = (heads[None, :] == jnp.arange(h, dtype=jnp.int32)[:, None]
              ).astype(jnp.float32)
    msg = _messages(ex, jnp.take(den, dst, axis=0),
                    jnp.take(feat, src, axis=0), expand)
    agg = jax.ops.segment_sum(msg, dst, num_segments=n)
    return _bias_relu(agg, b)


def kernel(x, edge_index, W1, al1, ar1, b1, W2, al2, ar2, b2):
    src = edge_index[0].astype(jnp.int32)
    dst = edge_index[1].astype(jnp.int32)
    n, in_feat = x.shape
    h1, d1 = al1.shape
    d2 = al2.shape[1]
    zeros_in = jnp.zeros((in_feat,), dtype=jnp.float32)
    hid = _gat_layer(x, src, dst, W1, al1, ar1, b1, h1, d1, zeros_in, False)
    out = _gat_layer(hid, src, dst, W2, al2, ar2, b2, 1, d2,
                     jnp.zeros((h1 * d1,), dtype=jnp.float32), False)
    return out


# speedup vs baseline: 2.0744x; 1.0211x over previous
import functools

import jax
import jax.numpy as jnp
from jax.experimental import pallas as pl

_N = 10000
_E = 160000
_NODE_TILE = 400
_EDGE_TILE = 1600


def _proj_body(x_ref, b_ref, w_ref, a_ref, feat_ref, eler_ref, *, apply_relu):
    xb = x_ref[...] + b_ref[...]
    if apply_relu:
        xb = jnp.maximum(xb, 0.0)
    feat = jnp.dot(xb, w_ref[...], preferred_element_type=jnp.float32)
    feat_ref[...] = feat.astype(jnp.bfloat16)
    eler_ref[...] = jnp.dot(feat, a_ref[...], preferred_element_type=jnp.float32)


def _proj(x, b, w, a_mat, apply_relu):
    n, k = x.shape
    m = w.shape[1]
    na = a_mat.shape[1]
    grid = n // _NODE_TILE
    return pl.pallas_call(
        functools.partial(_proj_body, apply_relu=apply_relu),
        grid=(grid,),
        in_specs=[
            pl.BlockSpec((_NODE_TILE, k), lambda i: (i, 0)),
            pl.BlockSpec((1, k), lambda i: (0, 0)),
            pl.BlockSpec((k, m), lambda i: (0, 0)),
            pl.BlockSpec((m, na), lambda i: (0, 0)),
        ],
        out_specs=[
            pl.BlockSpec((_NODE_TILE, m), lambda i: (i, 0)),
            pl.BlockSpec((_NODE_TILE, na), lambda i: (i, 0)),
        ],
        out_shape=[
            jax.ShapeDtypeStruct((n, m), jnp.bfloat16),
            jax.ShapeDtypeStruct((n, na), jnp.float32),
        ],
    )(x, b.reshape(1, k), w, a_mat)


def _edge_logit_body(els_ref, erd_ref, e_ref):
    s = els_ref[...] + erd_ref[...]
    e_ref[...] = jnp.where(s >= 0.0, s, 0.2 * s)


def _edge_exp_body(e_ref, m_ref, ex_ref):
    m = m_ref[...]
    m = jnp.where(jnp.isfinite(m), m, 0.0)
    ex_ref[...] = jnp.exp(e_ref[...] - m)


def _edge_ew(body, a, b):
    e, h = a.shape
    grid = e // _EDGE_TILE
    spec = pl.BlockSpec((_EDGE_TILE, h), lambda i: (i, 0))
    return pl.pallas_call(
        body,
        grid=(grid,),
        in_specs=[spec, spec],
        out_specs=spec,
        out_shape=jax.ShapeDtypeStruct((e, h), jnp.float32),
    )(a, b)


def _msg_body(ex_ref, den_ref, fs_ref, exp_ref, msg_ref):
    alpha = ex_ref[...] / jnp.maximum(den_ref[...], 1e-9)
    a_full = jnp.dot(alpha, exp_ref[...], preferred_element_type=jnp.float32)
    msg_ref[...] = a_full * fs_ref[...].astype(jnp.float32)


def _messages(ex, den_dst, feat_src, expand):
    e, h = ex.shape
    m = feat_src.shape[1]
    grid = e // _EDGE_TILE
    small = pl.BlockSpec((_EDGE_TILE, h), lambda i: (i, 0))
    big = pl.BlockSpec((_EDGE_TILE, m), lambda i: (i, 0))
    return pl.pallas_call(
        _msg_body,
        grid=(grid,),
        in_specs=[small, small, big, pl.BlockSpec((h, m), lambda i: (0, 0))],
        out_specs=big,
        out_shape=jax.ShapeDtypeStruct((e, m), jnp.float32),
    )(ex, den_dst, feat_src, expand)


def _bias_relu_body(x_ref, b_ref, o_ref):
    o_ref[...] = jnp.maximum(x_ref[...] + b_ref[...], 0.0)


def _bias_relu(x, b):
    n, m = x.shape
    grid = n // _NODE_TILE
    return pl.pallas_call(
        _bias_relu_body,
        grid=(grid,),
        in_specs=[
            pl.BlockSpec((_NODE_TILE, m), lambda i: (i, 0)),
            pl.BlockSpec((1, m), lambda i: (0, 0)),
        ],
        out_specs=pl.BlockSpec((_NODE_TILE, m), lambda i: (i, 0)),
        out_shape=jax.ShapeDtypeStruct((n, m), jnp.float32),
    )(x, b.reshape(1, m))


def _gat_layer(x, src, dst, w, al, ar, b, h, d, in_bias, in_relu):
    n = x.shape[0]
    hd = h * d
    a_mat = jnp.zeros((hd, 2 * h), dtype=jnp.float32)
    heads = jnp.arange(hd, dtype=jnp.int32) // d
    a_mat = a_mat.at[jnp.arange(hd), heads].set(al.reshape(-1))
    a_mat = a_mat.at[jnp.arange(hd), h + heads].set(ar.reshape(-1))
    feat, eler = _proj(x, in_bias, w, a_mat, in_relu)
    el = eler[:, :h]
    er = eler[:, h:]

    e = _edge_ew(_edge_logit_body, jnp.take(el, src, axis=0),
                 jnp.take(er, dst, axis=0))
    m = jax.ops.segment_max(e, dst, num_segments=n)
    ex = _edge_ew(_edge_exp_body, e, jnp.take(m, dst, axis=0))
    den = jax.ops.segment_sum(ex, dst, num_segments=n)

    expand = (heads[None, :] == jnp.arange(h, dtype=jnp.int32)[:, None]
              ).astype(jnp.float32)
    msg = _messages(ex, jnp.take(den, dst, axis=0),
                    jnp.take(feat, src, axis=0), expand)
    agg = jax.ops.segment_sum(msg, dst, num_segments=n)
    return _bias_relu(agg, b)


def kernel(x, edge_index, W1, al1, ar1, b1, W2, al2, ar2, b2):
    src = edge_index[0].astype(jnp.int32)
    dst = edge_index[1].astype(jnp.int32)
    n, in_feat = x.shape
    h1, d1 = al1.shape
    d2 = al2.shape[1]
    zeros_in = jnp.zeros((in_feat,), dtype=jnp.float32)
    hid = _gat_layer(x, src, dst, W1, al1, ar1, b1, h1, d1, zeros_in, False)
    out = _gat_layer(hid, src, dst, W2, al2, ar2, b2, 1, d2,
                     jnp.zeros((h1 * d1,), dtype=jnp.float32), False)
    return out
